# baseline (speedup 1.0000x reference)
import functools

import jax
import jax.numpy as jnp
from jax import lax
from jax.experimental import pallas as pl
from jax.experimental.pallas import tpu as pltpu
from jax.experimental.pallas import tpu_sc as plsc

_DIS_NORM = 6.0 * 1.0 * 512.0 / 1000.0
_D = 128
_P = _D * _D * _D
_HALF = _P // 2
_NS = 16
_NC = 2
_CHUNK = _P // _NS
_PP = 2048
_NPIECE = _CHUNK // _PP
_NIT = _PP // 16
_NR = _PP * 4 // 128
_PAD = 8192
_MTOT = _HALF + _PAD
_DUMMY = _HALF
_ZB = _MTOT // _NS // 8


def _mean_offsets(x3):
    nblk = 16
    bp = _P // nblk

    def body(x_ref, o_ref):
        g = pl.program_id(0)

        @pl.when(g == 0)
        def _():
            o_ref[...] = jnp.zeros_like(o_ref)

        s = jnp.sum(x_ref[...], axis=1)
        o_ref[...] += jnp.broadcast_to(s[:, None], o_ref.shape)

        @pl.when(g == nblk - 1)
        def _():
            o_ref[...] = 0.5 - o_ref[...] * (_DIS_NORM / _P)

    return pl.pallas_call(
        body,
        grid=(nblk,),
        in_specs=[pl.BlockSpec((3, bp), lambda g: (0, g))],
        out_specs=pl.BlockSpec((3, 16), lambda g: (0, 0)),
        out_shape=jax.ShapeDtypeStruct((3, 16), jnp.float32),
    )(x3)


def _floorfrac(pos):
    t = pos.astype(jnp.int32)
    tf = t.astype(jnp.float32)
    adj = tf > pos
    fl = t - jnp.where(adj, 1, 0)
    fr = pos - tf + jnp.where(adj, 1.0, 0.0)
    return fl, fr


def _paint(x4, offs):
    mesh = plsc.VectorSubcoreMesh(core_axis_name="c", subcore_axis_name="s")

    @functools.partial(
        pl.kernel,
        out_type=jax.ShapeDtypeStruct((_NC, _HALF), jnp.float32),
        mesh=mesh,
        scratch_types=[
            pltpu.VMEM((3, 16), jnp.float32),
            pltpu.VMEM((2, 4, _PP), jnp.float32),
            pltpu.VMEM((2, _NR, 128), jnp.int32),
            pltpu.VMEM((2, _NR, 128), jnp.float32),
            pltpu.VMEM((_ZB,), jnp.float32),
            pltpu.VMEM_SHARED((_MTOT,), jnp.float32),
            pltpu.SemaphoreType.DMA,
            pltpu.SemaphoreType.DMA,
        ],
    )
    def k(x_hbm, offs_hbm, out_hbm, offs_v, in_buf, idx_buf, val_buf,
          zero_buf, mesh_sh, in_sem, sc_sem):
        c = lax.axis_index("c")
        s = lax.axis_index("s")

        zero16 = jnp.zeros((16,), jnp.float32)

        def zb_body(i, carry):
            zero_buf[pl.ds(i * 16, 16)] = zero16
            return carry

        lax.fori_loop(0, _ZB // 16, zb_body, 0)
        zslice = _MTOT // _NS

        def zcp(i, carry):
            pltpu.sync_copy(zero_buf,
                            mesh_sh.at[pl.ds(s * zslice + i * _ZB, _ZB)])
            return carry

        lax.fori_loop(0, 8, zcp, 0)
        pltpu.sync_copy(offs_hbm, offs_v)
        plsc.subcore_barrier()

        offx = offs_v[0, :]
        offy = offs_v[1, :]
        offz = offs_v[2, :]
        lanes = lax.iota(jnp.int32, 16)
        lanes_f = lanes.astype(jnp.float32)
        ozl = offz + lanes_f
        base0 = s * _CHUNK

        def dma_in(pc, b):
            start = base0 + pc * _PP
            for ch in range(4):
                pltpu.async_copy(x_hbm.at[ch, pl.ds(start, _PP)],
                                 in_buf.at[b, ch], in_sem)

        def wait_in(b):
            for ch in range(4):
                pltpu.make_async_copy(x_hbm.at[ch, pl.ds(0, _PP)],
                                      in_buf.at[b, ch], in_sem).wait()

        def fire_rows(b, r0, nrows):
            def srow(r, c2):
                pltpu.async_copy(val_buf.at[b, r],
                                 mesh_sh.at[idx_buf.at[b, r]],
                                 sc_sem, add=True)
                return c2

            lax.fori_loop(r0, r0 + nrows, srow, 0)

        def wait_scatter():
            def wrow(r, c2):
                pltpu.make_async_copy(val_buf.at[0, 0],
                                      mesh_sh.at[idx_buf.at[0, 0]],
                                      sc_sem).wait()
                return c2

            lax.fori_loop(0, _NR, wrow, 0)

        def piece(pc, b):
            wait_in(b)

            @pl.when(pc + 1 < _NPIECE)
            def _():
                dma_in(pc + 1, 1 - b)

            @pl.when(pc >= 2)
            def _():
                wait_scatter()
            start = base0 + pc * _PP
            srow0 = s * (_CHUNK // 128) + pc * (_PP // 128)
            gxo = (start >> 14).astype(jnp.float32) + offx

            def subgroup(g, gcarry):
                @plsc.parallel_loop(g * 16, g * 16 + 16, unroll=4)
                def compute(it):
                        dx = in_buf[b, 0, pl.ds(it * 16, 16)]
                        dy = in_buf[b, 1, pl.ds(it * 16, 16)]
                        dz = in_buf[b, 2, pl.ds(it * 16, 16)]
                        va = in_buf[b, 3, pl.ds(it * 16, 16)]
                        gj = ((srow0 + (it >> 3)) & 127).astype(jnp.float32)
                        k0 = (it & 7) * 16
                        px = dx * _DIS_NORM + gxo
                        py = dy * _DIS_NORM + (gj + offy)
                        pz = dz * _DIS_NORM + (ozl + k0.astype(jnp.float32))
                        doff = _DUMMY + ((start + it * 16) & (_PAD // 2 - 1))
                        dummy = doff + lanes
                        ix, fx = _floorfrac(px)
                        iy, fy = _floorfrac(py)
                        iz, fz = _floorfrac(pz)
                        ox = 1.0 - fx
                        oy = 1.0 - fy
                        oz = 1.0 - fz
                        ko = (iz & 1) ^ c
                        kc = iz + ko
                        wz = jnp.where(ko == 0, oz, fz)
                        vw = va * wz
                        mx = (ix.astype(jnp.uint32) < 128,
                              (ix + 1).astype(jnp.uint32) < 128)
                        my = (iy.astype(jnp.uint32) < 128,
                              (iy + 1).astype(jnp.uint32) < 128)
                        mz = kc.astype(jnp.uint32) < 128
                        ax0 = ix * 8192
                        ax = (ax0, ax0 + 8192)
                        by0 = iy * 64 + (kc >> 1)
                        by = (by0, by0 + 64)
                        wx = (ox, fx)
                        wy = (oy, fy)
                        for di in range(2):
                            for dj in range(2):
                                w = (wx[di] * wy[dj]) * vw
                                idx = ax[di] + by[dj]
                                m = (mx[di] & my[dj]) & mz
                                idxs = jnp.where(m, idx, dummy)
                                n = di * 2 + dj
                                idx_buf[b, it >> 1,
                                        pl.ds((it & 1) * 64 + n * 16, 16)] = idxs
                                val_buf[b, it >> 1,
                                        pl.ds((it & 1) * 64 + n * 16, 16)] = w

                fire_rows(b, g * 8, 8)
                return gcarry

            lax.fori_loop(0, _NIT // 16, subgroup, 0)

        dma_in(0, 0)

        def piece_pair(q, carry):
            piece(q * 2, 0)
            piece(q * 2 + 1, 1)
            return carry

        lax.fori_loop(0, _NPIECE // 2, piece_pair, 0)
        wait_scatter()
        wait_scatter()
        plsc.subcore_barrier()

        oslice = _HALF // _NS
        pltpu.sync_copy(mesh_sh.at[pl.ds(s * oslice, oslice)],
                        out_hbm.at[c, pl.ds(s * oslice, oslice)])

    return k(x4, offs)


def kernel(x):
    x4 = x.reshape(4, _P)
    offs = _mean_offsets(x4[:3])
    mesh2 = _paint(x4, offs)
    out = mesh2.reshape(2, _D * _D, _D // 2).transpose(1, 2, 0)
    return out.reshape(1, 1, _D, _D, _D)


# speedup vs baseline: 1.0226x; 1.0226x over previous
import functools

import jax
import jax.numpy as jnp
from jax import lax
from jax.experimental import pallas as pl
from jax.experimental.pallas import tpu as pltpu
from jax.experimental.pallas import tpu_sc as plsc

_DIS_NORM = 6.0 * 1.0 * 512.0 / 1000.0
_D = 128
_P = _D * _D * _D
_HALF = _P // 2
_NS = 16
_NC = 2
_CHUNK = _P // _NS
_PP = 2048
_NPIECE = _CHUNK // _PP
_NIT = _PP // 16
_NR = _PP * 4 // 128
_PAD = 8192
_MTOT = _HALF + _PAD
_DUMMY = _HALF
_ZB = _MTOT // _NS // 8


def _mean_offsets(x3):
    nblk = 16
    bp = _P // nblk

    def body(x_ref, o_ref):
        g = pl.program_id(0)

        @pl.when(g == 0)
        def _():
            o_ref[...] = jnp.zeros_like(o_ref)

        s = jnp.sum(x_ref[...], axis=1)
        o_ref[...] += jnp.broadcast_to(s[:, None], o_ref.shape)

        @pl.when(g == nblk - 1)
        def _():
            o_ref[...] = 0.5 - o_ref[...] * (_DIS_NORM / _P)

    return pl.pallas_call(
        body,
        grid=(nblk,),
        in_specs=[pl.BlockSpec((3, bp), lambda g: (0, g))],
        out_specs=pl.BlockSpec((3, 16), lambda g: (0, 0)),
        out_shape=jax.ShapeDtypeStruct((3, 16), jnp.float32),
    )(x3)


def _floorfrac(pos):
    t = pos.astype(jnp.int32)
    tf = t.astype(jnp.float32)
    adj = tf > pos
    fl = t - jnp.where(adj, 1, 0)
    fr = pos - tf + jnp.where(adj, 1.0, 0.0)
    return fl, fr


def _paint(x4, offs):
    mesh = plsc.VectorSubcoreMesh(core_axis_name="c", subcore_axis_name="s")

    @functools.partial(
        pl.kernel,
        out_type=jax.ShapeDtypeStruct((_NC, _HALF), jnp.float32),
        mesh=mesh,
        scratch_types=[
            pltpu.VMEM((3, 16), jnp.float32),
            pltpu.VMEM((2, 4, _PP), jnp.float32),
            pltpu.VMEM((2, _NR, 128), jnp.int32),
            pltpu.VMEM((2, _NR, 128), jnp.float32),
            pltpu.VMEM((_ZB,), jnp.float32),
            pltpu.VMEM_SHARED((_MTOT,), jnp.float32),
            pltpu.SemaphoreType.DMA,
            pltpu.SemaphoreType.DMA,
        ],
    )
    def k(x_hbm, offs_hbm, out_hbm, offs_v, in_buf, idx_buf, val_buf,
          zero_buf, mesh_sh, in_sem, sc_sem):
        c = lax.axis_index("c")
        s = lax.axis_index("s")

        zero16 = jnp.zeros((16,), jnp.float32)

        def zb_body(i, carry):
            zero_buf[pl.ds(i * 16, 16)] = zero16
            return carry

        lax.fori_loop(0, _ZB // 16, zb_body, 0)
        zslice = _MTOT // _NS

        def zcp(i, carry):
            pltpu.sync_copy(zero_buf,
                            mesh_sh.at[pl.ds(s * zslice + i * _ZB, _ZB)])
            return carry

        lax.fori_loop(0, 8, zcp, 0)
        pltpu.sync_copy(offs_hbm, offs_v)
        plsc.subcore_barrier()

        offx = offs_v[0, :]
        offy = offs_v[1, :]
        offz = offs_v[2, :]
        lanes = lax.iota(jnp.int32, 16)
        lanes_f = lanes.astype(jnp.float32)
        ozl = offz + lanes_f
        base0 = s * _CHUNK

        def dma_in(pc, b):
            start = base0 + pc * _PP
            for ch in range(4):
                pltpu.async_copy(x_hbm.at[ch, pl.ds(start, _PP)],
                                 in_buf.at[b, ch], in_sem)

        def wait_in(b):
            for ch in range(4):
                pltpu.make_async_copy(x_hbm.at[ch, pl.ds(0, _PP)],
                                      in_buf.at[b, ch], in_sem).wait()

        def fire_rows(b, r0, nrows):
            def srow(r, c2):
                pltpu.async_copy(val_buf.at[b, r],
                                 mesh_sh.at[idx_buf.at[b, r]],
                                 sc_sem, add=True)
                return c2

            lax.fori_loop(r0, r0 + nrows, srow, 0)

        def wait_rows(nrows):
            def wrow(r, c2):
                pltpu.make_async_copy(val_buf.at[0, 0],
                                      mesh_sh.at[idx_buf.at[0, 0]],
                                      sc_sem).wait()
                return c2

            lax.fori_loop(0, nrows, wrow, 0)

        def piece(pc, b):
            wait_in(b)

            @pl.when(pc + 1 < _NPIECE)
            def _():
                dma_in(pc + 1, 1 - b)

            start = base0 + pc * _PP
            srow0 = s * (_CHUNK // 128) + pc * (_PP // 128)
            gxo = (start >> 14).astype(jnp.float32) + offx

            def subgroup(g, gcarry):
                @plsc.parallel_loop(g * 16, g * 16 + 16, unroll=2)
                def compute(it):
                        dx = in_buf[b, 0, pl.ds(it * 16, 16)]
                        dy = in_buf[b, 1, pl.ds(it * 16, 16)]
                        dz = in_buf[b, 2, pl.ds(it * 16, 16)]
                        va = in_buf[b, 3, pl.ds(it * 16, 16)]
                        gj = ((srow0 + (it >> 3)) & 127).astype(jnp.float32)
                        k0 = (it & 7) * 16
                        px = dx * _DIS_NORM + gxo
                        py = dy * _DIS_NORM + (gj + offy)
                        pz = dz * _DIS_NORM + (ozl + k0.astype(jnp.float32))
                        doff = _DUMMY + ((start + it * 16) & (_PAD // 2 - 1))
                        dummy = doff + lanes
                        ix, fx = _floorfrac(px)
                        iy, fy = _floorfrac(py)
                        iz, fz = _floorfrac(pz)
                        ox = 1.0 - fx
                        oy = 1.0 - fy
                        oz = 1.0 - fz
                        ko = (iz & 1) ^ c
                        kc = iz + ko
                        wz = jnp.where(ko == 0, oz, fz)
                        vw = va * wz
                        mx = (ix.astype(jnp.uint32) < 128,
                              (ix + 1).astype(jnp.uint32) < 128)
                        my = (iy.astype(jnp.uint32) < 128,
                              (iy + 1).astype(jnp.uint32) < 128)
                        mz = kc.astype(jnp.uint32) < 128
                        ax0 = ix * 8192
                        ax = (ax0, ax0 + 8192)
                        by0 = iy * 64 + (kc >> 1)
                        by = (by0, by0 + 64)
                        wx = (ox * vw, fx * vw)
                        wy = (oy, fy)
                        mxz = (mx[0] & mz, mx[1] & mz)
                        for di in range(2):
                            for dj in range(2):
                                w = wx[di] * wy[dj]
                                idx = ax[di] + by[dj]
                                m = mxz[di] & my[dj]
                                idxs = jnp.where(m, idx, dummy)
                                n = di * 2 + dj
                                idx_buf[b, it >> 1,
                                        pl.ds((it & 1) * 64 + n * 16, 16)] = idxs
                                val_buf[b, it >> 1,
                                        pl.ds((it & 1) * 64 + n * 16, 16)] = w

                @pl.when(pc >= 2)
                def _():
                    wait_rows(8)
                fire_rows(b, g * 8, 8)
                return gcarry

            lax.fori_loop(0, _NIT // 16, subgroup, 0)

        dma_in(0, 0)

        def piece_pair(q, carry):
            piece(q * 2, 0)
            piece(q * 2 + 1, 1)
            return carry

        lax.fori_loop(0, _NPIECE // 2, piece_pair, 0)
        wait_rows(2 * _NR)
        plsc.subcore_barrier()

        oslice = _HALF // _NS
        pltpu.sync_copy(mesh_sh.at[pl.ds(s * oslice, oslice)],
                        out_hbm.at[c, pl.ds(s * oslice, oslice)])

    return k(x4, offs)


def kernel(x):
    x4 = x.reshape(4, _P)
    offs = _mean_offsets(x4[:3])
    mesh2 = _paint(x4, offs)
    out = mesh2.reshape(2, _D * _D, _D // 2).transpose(1, 2, 0)
    return out.reshape(1, 1, _D, _D, _D)
